# baseline (speedup 1.0000x reference)
import functools

import jax
import jax.numpy as jnp
from jax import lax
from jax.experimental import pallas as pl
from jax.experimental.pallas import tpu as pltpu
from jax.experimental.pallas import tpu_sc as plsc

BATCH = 4
NUM_POSITIONS = 8192
FEATURE_DIM = 768

_NC = 2
_NS = 16
_NW = _NC * _NS
_SPAN = NUM_POSITIONS // _NW
_C = 32
_CHUNKS = _SPAN // _C
_CW = _C * FEATURE_DIM


def _sc_body(x_hbm, e_hbm, o_hbm, ebuf, xbuf):
    wid = lax.axis_index("s") * _NC + lax.axis_index("c")
    pos_base = wid * _SPAN

    for chunk in range(_CHUNKS):
        pos0 = pos_base + chunk * _C
        pltpu.sync_copy(e_hbm.at[pl.ds(pos0 * FEATURE_DIM, _CW)], ebuf)
        for b in range(BATCH):
            off0 = (b * NUM_POSITIONS + pos0) * FEATURE_DIM
            pltpu.sync_copy(x_hbm.at[pl.ds(off0, _CW)], xbuf)

            @plsc.parallel_loop(0, _CW, 16, unroll=8)
            def _add(i):
                xbuf[pl.ds(i, 16)] = xbuf[pl.ds(i, 16)] + ebuf[pl.ds(i, 16)]

            pltpu.sync_copy(xbuf, o_hbm.at[pl.ds(off0, _CW)])


@functools.partial(
    pl.kernel,
    out_type=jax.ShapeDtypeStruct((BATCH * NUM_POSITIONS * FEATURE_DIM,), jnp.float32),
    mesh=plsc.VectorSubcoreMesh(core_axis_name="c", subcore_axis_name="s"),
    scratch_types=[
        pltpu.VMEM((_CW,), jnp.float32),
        pltpu.VMEM((_CW,), jnp.float32),
    ],
)
def _sc_kernel(x_hbm, e_hbm, o_hbm, ebuf, xbuf):
    _sc_body(x_hbm, e_hbm, o_hbm, ebuf, xbuf)


def kernel(x, embedding):
    x1d = x.reshape(-1)
    e1d = embedding.reshape(-1)
    out = _sc_kernel(x1d, e1d)
    return out.reshape(BATCH, NUM_POSITIONS, FEATURE_DIM)


# speedup vs baseline: 1.2057x; 1.2057x over previous
import functools

import jax
import jax.numpy as jnp
from jax import lax
from jax.experimental import pallas as pl
from jax.experimental.pallas import tpu as pltpu
from jax.experimental.pallas import tpu_sc as plsc

BATCH = 4
NUM_POSITIONS = 8192
FEATURE_DIM = 768

_NC = 2
_NS = 16
_NW = _NC * _NS
_SPAN = NUM_POSITIONS // _NW
_C = 16
_CHUNKS = _SPAN // _C
_CW = _C * FEATURE_DIM
_STEPS = _CHUNKS * BATCH


def _sc_body(x_hbm, e_hbm, o_hbm, xbufs, obufs, ebufs, lsems, ssems, esems):
    wid = lax.axis_index("s") * _NC + lax.axis_index("c")
    pos_base = wid * _SPAN

    def e_off(chunk):
        return (pos_base + chunk * _C) * FEATURE_DIM

    def xo_off(step):
        chunk, b = divmod(step, BATCH)
        return (b * NUM_POSITIONS + pos_base + chunk * _C) * FEATURE_DIM

    def e_load(chunk):
        return pltpu.async_copy(
            e_hbm.at[pl.ds(e_off(chunk), _CW)], ebufs[chunk % 2], esems[chunk % 2])

    def x_load(step):
        return pltpu.async_copy(
            x_hbm.at[pl.ds(xo_off(step), _CW)], xbufs[step % 2], lsems[step % 2])

    def o_store(step):
        return pltpu.async_copy(
            obufs[step % 2], o_hbm.at[pl.ds(xo_off(step), _CW)], ssems[step % 2])

    loads = {0: x_load(0)}
    e_loads = {0: e_load(0)}
    stores = {}
    for s in range(_STEPS):
        chunk, b = divmod(s, BATCH)
        p = s % 2
        if s + 1 < _STEPS:
            loads[s + 1] = x_load(s + 1)
        if b == 0 and chunk + 1 < _CHUNKS:
            e_loads[chunk + 1] = e_load(chunk + 1)
        loads.pop(s).wait()
        if b == 0:
            e_loads[chunk].wait()
        if s >= 2:
            stores.pop(s - 2).wait()
        xbuf, obuf, ebuf = xbufs[p], obufs[p], ebufs[chunk % 2]

        @plsc.parallel_loop(0, _CW, 16, unroll=8)
        def _add(i):
            obuf[pl.ds(i, 16)] = xbuf[pl.ds(i, 16)] + ebuf[pl.ds(i, 16)]

        stores[s] = o_store(s)
    for s in list(stores):
        stores.pop(s).wait()


@functools.partial(
    pl.kernel,
    out_type=jax.ShapeDtypeStruct((BATCH * NUM_POSITIONS * FEATURE_DIM,), jnp.float32),
    mesh=plsc.VectorSubcoreMesh(core_axis_name="c", subcore_axis_name="s"),
    scratch_types=[
        [pltpu.VMEM((_CW,), jnp.float32) for _ in range(2)],
        [pltpu.VMEM((_CW,), jnp.float32) for _ in range(2)],
        [pltpu.VMEM((_CW,), jnp.float32) for _ in range(2)],
        [pltpu.SemaphoreType.DMA for _ in range(2)],
        [pltpu.SemaphoreType.DMA for _ in range(2)],
        [pltpu.SemaphoreType.DMA for _ in range(2)],
    ],
)
def _sc_kernel(x_hbm, e_hbm, o_hbm, xbufs, obufs, ebufs, lsems, ssems, esems):
    _sc_body(x_hbm, e_hbm, o_hbm, xbufs, obufs, ebufs, lsems, ssems, esems)


def kernel(x, embedding):
    x1d = x.reshape(-1)
    e1d = embedding.reshape(-1)
    out = _sc_kernel(x1d, e1d)
    return out.reshape(BATCH, NUM_POSITIONS, FEATURE_DIM)


# speedup vs baseline: 3.9169x; 3.2487x over previous
import functools

import jax
import jax.numpy as jnp
from jax import lax
from jax.experimental import pallas as pl
from jax.experimental.pallas import tpu as pltpu
from jax.experimental.pallas import tpu_sc as plsc

BATCH = 4
NUM_POSITIONS = 8192
FEATURE_DIM = 768

_NC = 2
_NS = 16
_NW = _NC * _NS
_SPAN = NUM_POSITIONS // _NW
_C = 16
_CHUNKS = _SPAN // _C
_XROWS = BATCH * NUM_POSITIONS


def _sc_body(x_hbm, e_hbm, o_hbm, xbufs, obufs, ebufs, lsems, ssems, esems):
    wid = lax.axis_index("s") * _NC + lax.axis_index("c")
    pos_base = wid * _SPAN

    def e_load(chunk, cc):
        row = jnp.minimum(pos_base + chunk * _C, NUM_POSITIONS - _C)
        return pltpu.async_copy(e_hbm.at[pl.ds(row, _C)], ebufs[cc], esems[cc])

    def x_load(chunk, b):
        row = jnp.minimum(b * NUM_POSITIONS + pos_base + chunk * _C, _XROWS - _C)
        return pltpu.async_copy(x_hbm.at[pl.ds(row, _C)], xbufs[b], lsems[b])

    def o_store(chunk, b):
        row = b * NUM_POSITIONS + pos_base + chunk * _C
        return pltpu.async_copy(obufs[b % 2], o_hbm.at[pl.ds(row, _C)], ssems[b % 2])

    def wait_x(b):
        pltpu.make_async_copy(x_hbm.at[pl.ds(0, _C)], xbufs[b], lsems[b]).wait()

    def wait_e(cc):
        pltpu.make_async_copy(e_hbm.at[pl.ds(0, _C)], ebufs[cc], esems[cc]).wait()

    def wait_s(j):
        pltpu.make_async_copy(obufs[j], o_hbm.at[pl.ds(0, _C)], ssems[j]).wait()

    e_load(0, 0)
    e_load(1, 1)
    for b in range(BATCH):
        x_load(0, b)
    o_store(0, 0)
    o_store(0, 1)

    def pair_body(k, carry):
        for cc in range(2):
            c = 2 * k + cc
            wait_e(cc)
            for b in range(BATCH):
                wait_x(b)
                wait_s(b % 2)
                xbuf, obuf, ebuf = xbufs[b], obufs[b % 2], ebufs[cc]

                @plsc.parallel_loop(0, FEATURE_DIM, 16)
                def _add(i):
                    for r in range(_C):
                        obuf[r, pl.ds(i, 16)] = (
                            xbuf[r, pl.ds(i, 16)] + ebuf[r, pl.ds(i, 16)])

                x_load(c + 1, b)
                o_store(c, b)
            e_load(c + 2, cc)
        return carry

    lax.fori_loop(0, _CHUNKS // 2, pair_body, 0)

    for b in range(BATCH):
        wait_x(b)
    wait_e(0)
    wait_e(1)
    wait_s(0)
    wait_s(1)


@functools.partial(
    pl.kernel,
    out_type=jax.ShapeDtypeStruct((BATCH * NUM_POSITIONS, FEATURE_DIM), jnp.float32),
    mesh=plsc.VectorSubcoreMesh(core_axis_name="c", subcore_axis_name="s"),
    scratch_types=[
        [pltpu.VMEM((_C, FEATURE_DIM), jnp.float32) for _ in range(BATCH)],
        [pltpu.VMEM((_C, FEATURE_DIM), jnp.float32) for _ in range(2)],
        [pltpu.VMEM((_C, FEATURE_DIM), jnp.float32) for _ in range(2)],
        [pltpu.SemaphoreType.DMA for _ in range(BATCH)],
        [pltpu.SemaphoreType.DMA for _ in range(2)],
        [pltpu.SemaphoreType.DMA for _ in range(2)],
    ],
)
def _sc_kernel(x_hbm, e_hbm, o_hbm, xbufs, obufs, ebufs, lsems, ssems, esems):
    _sc_body(x_hbm, e_hbm, o_hbm, xbufs, obufs, ebufs, lsems, ssems, esems)


def kernel(x, embedding):
    x2d = x.reshape(BATCH * NUM_POSITIONS, FEATURE_DIM)
    out = _sc_kernel(x2d, embedding)
    return out.reshape(BATCH, NUM_POSITIONS, FEATURE_DIM)
